# baseline (speedup 1.0000x reference)
import jax
import jax.numpy as jnp
from jax.experimental import pallas as pl
from jax.experimental.pallas import tpu as pltpu

N = 8192
INPUT_D = 256
HIDDEN_D = 128
OUTPUT_D = 64
ALP = 0.5
LAM = 1.0

BM = 512
BK = 4096


def _h_kernel(x_ref, w_ref, b_ref, h_ref):
    h_ref[...] = (
        jnp.dot(x_ref[...], w_ref[...], preferred_element_type=jnp.float32)
        + b_ref[...]
    )


def _pass1_kernel(a_ref, d_ref, hk_ref, hi_ref, y1_ref, dh_ref, acc_a, acc_d):
    k = pl.program_id(1)

    @pl.when(k == 0)
    def _():
        acc_a[...] = jnp.zeros_like(acc_a)
        acc_d[...] = jnp.zeros_like(acc_d)

    hk = hk_ref[...]
    acc_a[...] += jnp.dot(a_ref[...], hk, preferred_element_type=jnp.float32)
    acc_d[...] += jnp.dot(d_ref[...], hk, preferred_element_type=jnp.float32)

    @pl.when(k == pl.num_programs(1) - 1)
    def _():
        dh = acc_d[...]
        dh_ref[...] = dh
        y1_ref[...] = (1.0 - ALP) * hi_ref[...] + (ALP * LAM) * acc_a[...] + ALP * dh


def _pass2_kernel(a_ref, yk_ref, yi_ref, dh_ref, w_ref, b_ref, out_ref, acc):
    k = pl.program_id(1)

    @pl.when(k == 0)
    def _():
        acc[...] = jnp.zeros_like(acc)

    acc[...] += jnp.dot(a_ref[...], yk_ref[...], preferred_element_type=jnp.float32)

    @pl.when(k == pl.num_programs(1) - 1)
    def _():
        y2 = (
            (1.0 - ALP) * yi_ref[...]
            + (ALP * LAM) * acc[...]
            + ALP * dh_ref[...]
        )
        z = jnp.maximum(y2, 0.0)
        out_ref[...] = (
            jnp.dot(z, w_ref[...], preferred_element_type=jnp.float32) + b_ref[...]
        )


def kernel(x, sem_adj, norm_diag, W_bef, b_bef, W_aft, b_aft):
    h = pl.pallas_call(
        _h_kernel,
        out_shape=jax.ShapeDtypeStruct((N, HIDDEN_D), jnp.float32),
    )(x, W_bef, b_bef.reshape(1, HIDDEN_D))

    grid = (N // BM, N // BK)
    y1, dh = pl.pallas_call(
        _pass1_kernel,
        grid=grid,
        in_specs=[
            pl.BlockSpec((BM, BK), lambda i, k: (i, k)),
            pl.BlockSpec((BM, BK), lambda i, k: (i, k)),
            pl.BlockSpec((BK, HIDDEN_D), lambda i, k: (k, 0)),
            pl.BlockSpec((BM, HIDDEN_D), lambda i, k: (i, 0)),
        ],
        out_specs=[
            pl.BlockSpec((BM, HIDDEN_D), lambda i, k: (i, 0)),
            pl.BlockSpec((BM, HIDDEN_D), lambda i, k: (i, 0)),
        ],
        out_shape=[
            jax.ShapeDtypeStruct((N, HIDDEN_D), jnp.float32),
            jax.ShapeDtypeStruct((N, HIDDEN_D), jnp.float32),
        ],
        scratch_shapes=[
            pltpu.VMEM((BM, HIDDEN_D), jnp.float32),
            pltpu.VMEM((BM, HIDDEN_D), jnp.float32),
        ],
        compiler_params=pltpu.CompilerParams(
            dimension_semantics=("parallel", "arbitrary"),
        ),
    )(sem_adj, norm_diag, h, h)

    out = pl.pallas_call(
        _pass2_kernel,
        grid=grid,
        in_specs=[
            pl.BlockSpec((BM, BK), lambda i, k: (i, k)),
            pl.BlockSpec((BK, HIDDEN_D), lambda i, k: (k, 0)),
            pl.BlockSpec((BM, HIDDEN_D), lambda i, k: (i, 0)),
            pl.BlockSpec((BM, HIDDEN_D), lambda i, k: (i, 0)),
            pl.BlockSpec((HIDDEN_D, OUTPUT_D), lambda i, k: (0, 0)),
            pl.BlockSpec((1, OUTPUT_D), lambda i, k: (0, 0)),
        ],
        out_specs=pl.BlockSpec((BM, OUTPUT_D), lambda i, k: (i, 0)),
        out_shape=jax.ShapeDtypeStruct((N, OUTPUT_D), jnp.float32),
        scratch_shapes=[pltpu.VMEM((BM, HIDDEN_D), jnp.float32)],
        compiler_params=pltpu.CompilerParams(
            dimension_semantics=("parallel", "arbitrary"),
        ),
    )(sem_adj, y1, y1, dh, W_aft, b_aft.reshape(1, OUTPUT_D))

    return out


# speedup vs baseline: 1.1409x; 1.1409x over previous
import jax
import jax.numpy as jnp
from jax.experimental import pallas as pl
from jax.experimental.pallas import tpu as pltpu

N = 8192
INPUT_D = 256
HIDDEN_D = 128
OUTPUT_D = 64
ALP = 0.5
LAM = 1.0

BM1 = 256
BM2 = 512


def _pass1_kernel(x_ref, w_ref, b_ref, a_ref, d_ref, y1_ref, dh_ref, h_scr):
    i = pl.program_id(0)

    @pl.when(i == 0)
    def _():
        h_scr[...] = (
            jnp.dot(x_ref[...], w_ref[...], preferred_element_type=jnp.float32)
            + b_ref[...]
        )

    h = h_scr[...]
    ah = jnp.dot(a_ref[...], h, preferred_element_type=jnp.float32)
    dh = jnp.dot(d_ref[...], h, preferred_element_type=jnp.float32)
    hi = h_scr[pl.ds(i * BM1, BM1), :]
    dh_ref[...] = dh
    y1_ref[...] = (1.0 - ALP) * hi + (ALP * LAM) * ah + ALP * dh


def _pass2_kernel(a_ref, y1_ref, dh_ref, w_ref, b_ref, out_ref):
    i = pl.program_id(0)
    y1 = y1_ref[...]
    ay = jnp.dot(a_ref[...], y1, preferred_element_type=jnp.float32)
    yi = y1_ref[pl.ds(i * BM2, BM2), :]
    y2 = (1.0 - ALP) * yi + (ALP * LAM) * ay + ALP * dh_ref[...]
    z = jnp.maximum(y2, 0.0)
    out_ref[...] = (
        jnp.dot(z, w_ref[...], preferred_element_type=jnp.float32) + b_ref[...]
    )


def kernel(x, sem_adj, norm_diag, W_bef, b_bef, W_aft, b_aft):
    y1, dh = pl.pallas_call(
        _pass1_kernel,
        grid=(N // BM1,),
        in_specs=[
            pl.BlockSpec((N, INPUT_D), lambda i: (0, 0)),
            pl.BlockSpec((INPUT_D, HIDDEN_D), lambda i: (0, 0)),
            pl.BlockSpec((1, HIDDEN_D), lambda i: (0, 0)),
            pl.BlockSpec((BM1, N), lambda i: (i, 0)),
            pl.BlockSpec((BM1, N), lambda i: (i, 0)),
        ],
        out_specs=[
            pl.BlockSpec((BM1, HIDDEN_D), lambda i: (i, 0)),
            pl.BlockSpec((BM1, HIDDEN_D), lambda i: (i, 0)),
        ],
        out_shape=[
            jax.ShapeDtypeStruct((N, HIDDEN_D), jnp.float32),
            jax.ShapeDtypeStruct((N, HIDDEN_D), jnp.float32),
        ],
        scratch_shapes=[pltpu.VMEM((N, HIDDEN_D), jnp.float32)],
        compiler_params=pltpu.CompilerParams(
            dimension_semantics=("arbitrary",),
        ),
    )(x, W_bef, b_bef.reshape(1, HIDDEN_D), sem_adj, norm_diag)

    out = pl.pallas_call(
        _pass2_kernel,
        grid=(N // BM2,),
        in_specs=[
            pl.BlockSpec((BM2, N), lambda i: (i, 0)),
            pl.BlockSpec((N, HIDDEN_D), lambda i: (0, 0)),
            pl.BlockSpec((BM2, HIDDEN_D), lambda i: (i, 0)),
            pl.BlockSpec((HIDDEN_D, OUTPUT_D), lambda i: (0, 0)),
            pl.BlockSpec((1, OUTPUT_D), lambda i: (0, 0)),
        ],
        out_specs=pl.BlockSpec((BM2, OUTPUT_D), lambda i: (i, 0)),
        out_shape=jax.ShapeDtypeStruct((N, OUTPUT_D), jnp.float32),
        compiler_params=pltpu.CompilerParams(
            dimension_semantics=("parallel",),
        ),
    )(sem_adj, y1, dh, W_aft, b_aft.reshape(1, OUTPUT_D))

    return out


# speedup vs baseline: 1.1891x; 1.0423x over previous
import jax
import jax.numpy as jnp
from jax.experimental import pallas as pl
from jax.experimental.pallas import tpu as pltpu

N = 8192
INPUT_D = 256
HIDDEN_D = 128
OUTPUT_D = 64
ALP = 0.5
LAM = 1.0

BM = 256
P = N // BM


def _fused_kernel(
    x_ref, w1_ref, b1_ref, a_ref, d_ref, w2_ref, b2_ref,
    out_ref, h_scr, y1_scr, dh_scr,
):
    i = pl.program_id(0)

    @pl.when(i == 0)
    def _():
        h_scr[...] = (
            jnp.dot(x_ref[...], w1_ref[...], preferred_element_type=jnp.float32)
            + b1_ref[...]
        )

    @pl.when(i < P)
    def _():
        h = h_scr[...]
        ah = jnp.dot(a_ref[...], h, preferred_element_type=jnp.float32)
        dh = jnp.dot(d_ref[...], h, preferred_element_type=jnp.float32)
        rows = pl.ds(i * BM, BM)
        dh_scr[rows, :] = dh
        y1_scr[rows, :] = (1.0 - ALP) * h_scr[rows, :] + (ALP * LAM) * ah + ALP * dh

    @pl.when(i >= P)
    def _():
        j = i - P
        y1 = y1_scr[...]
        ay = jnp.dot(a_ref[...], y1, preferred_element_type=jnp.float32)
        rows = pl.ds(j * BM, BM)
        y2 = (
            (1.0 - ALP) * y1_scr[rows, :]
            + (ALP * LAM) * ay
            + ALP * dh_scr[rows, :]
        )
        z = jnp.maximum(y2, 0.0)
        out_ref[...] = (
            jnp.dot(z, w2_ref[...], preferred_element_type=jnp.float32)
            + b2_ref[...]
        )


def kernel(x, sem_adj, norm_diag, W_bef, b_bef, W_aft, b_aft):
    out = pl.pallas_call(
        _fused_kernel,
        grid=(2 * P,),
        in_specs=[
            pl.BlockSpec((N, INPUT_D), lambda i: (0, 0)),
            pl.BlockSpec((INPUT_D, HIDDEN_D), lambda i: (0, 0)),
            pl.BlockSpec((1, HIDDEN_D), lambda i: (0, 0)),
            pl.BlockSpec((BM, N), lambda i: (jnp.where(i < P, i, i - P), 0)),
            pl.BlockSpec((BM, N), lambda i: (jnp.minimum(i, P - 1), 0)),
            pl.BlockSpec((HIDDEN_D, OUTPUT_D), lambda i: (0, 0)),
            pl.BlockSpec((1, OUTPUT_D), lambda i: (0, 0)),
        ],
        out_specs=pl.BlockSpec(
            (BM, OUTPUT_D), lambda i: (jnp.maximum(i - P, 0), 0)
        ),
        out_shape=jax.ShapeDtypeStruct((N, OUTPUT_D), jnp.float32),
        scratch_shapes=[
            pltpu.VMEM((N, HIDDEN_D), jnp.float32),
            pltpu.VMEM((N, HIDDEN_D), jnp.float32),
            pltpu.VMEM((N, HIDDEN_D), jnp.float32),
        ],
        compiler_params=pltpu.CompilerParams(
            dimension_semantics=("arbitrary",),
        ),
    )(
        x, W_bef, b_bef.reshape(1, HIDDEN_D), sem_adj, norm_diag,
        W_aft, b_aft.reshape(1, OUTPUT_D),
    )
    return out
